# baseline (speedup 1.0000x reference)
import functools
import math

import jax
import jax.numpy as jnp
from jax import lax
from jax.experimental import pallas as pl
from jax.experimental.pallas import tpu as pltpu
from jax.experimental.pallas import tpu_sc as plsc

D = 256
NHEAD = 32
DH = D // NHEAD
NLAYERS = 3
VOCAB = 55
VPAD = 64
B, N, E = 32, 256, 510
EPAD = 512
XW = N * D
HW = D // 2
AGW = N * HW
UNROLL = 4


def _ln(x, s, b, eps=1e-5):
    mu = jnp.mean(x, axis=-1, keepdims=True)
    var = jnp.mean((x - mu) ** 2, axis=-1, keepdims=True)
    return (x - mu) / jnp.sqrt(var + eps) * s + b


def _attn_ffn(x, wq, wk, wv, bv, wo, w1, w2, l1s, l1b, l2s, l2b):
    q = (x @ wq).astype(jnp.bfloat16)
    k = (x @ wk).astype(jnp.bfloat16)
    v = (x @ wv + bv).astype(jnp.bfloat16)
    pairs = [(g, h) for g in range(G) for h in range(NHEAD)]
    outs = {}

    def _score(g, h):
        rows = slice(g * N, (g + 1) * N)
        sl = slice(h * DH, (h + 1) * DH)
        return lax.dot_general(q[rows, sl], k[rows, sl],
                               (((1,), (1,)), ((), ())),
                               preferred_element_type=jnp.float32)

    def _finish(g, h, s):
        rows = slice(g * N, (g + 1) * N)
        sl = slice(h * (DH + 1), (h + 1) * (DH + 1))
        p = jnp.exp2(s.astype(jnp.bfloat16))
        outs[(g, h)] = lax.dot_general(p, v[rows, sl],
                                       (((1,), (0,)), ((), ())),
                                       preferred_element_type=jnp.float32)

    prev = None
    for g, h in pairs:
        s = _score(g, h)
        if prev is not None:
            _finish(*prev)
        prev = (g, h, s)
    _finish(*prev)

    expander = (lax.broadcasted_iota(jnp.int32, (NHEAD, D), 0)
                == (lax.broadcasted_iota(jnp.int32, (NHEAD, D), 1) >> 3)
                ).astype(jnp.float32)
    gouts = []
    for g in range(G):
        og = jnp.concatenate(
            [outs[(g, h)][:, :DH] for h in range(NHEAD)], axis=1)
        sg = jnp.concatenate(
            [outs[(g, h)][:, DH:DH + 1] for h in range(NHEAD)], axis=1)
        gouts.append(og * ((1.0 / sg) @ expander))
    o = jnp.concatenate(gouts, axis=0) @ wo
    x = _ln(x + o, l1s, l1b)
    ff = jnp.maximum(x @ w1, 0.0) @ w2
    x = _ln(x + ff, l2s, l2b)
    return x


G = 2


def _body_first(lab_ref, emb_ref, wq, wk, wv, bv, wo, w1, w2, l1s, l1b, l2s,
                l2b, out_ref):
    iota = lax.broadcasted_iota(jnp.int32, (N, VPAD), 1)
    onehot = jnp.concatenate(
        [(lab_ref[0, g, :][:, None] == iota).astype(jnp.float32)
         for g in range(G)], axis=0)
    x = onehot @ emb_ref[...]
    out_ref[0] = _attn_ffn(x, wq[...], wk[...], wv[...], bv[...], wo[...],
                           w1[...], w2[...], l1s[...], l1b[...], l2s[...],
                           l2b[...]).reshape(G, N, D)


def _body_mid(x_ref, a0_ref, a1_ref, wg, bg, wq, wk, wv, bv, wo, w1, w2, l1s,
              l1b, l2s, l2b, out_ref):
    agg = jnp.concatenate([a0_ref[0], a1_ref[0]], axis=-1).reshape(G * N, D)
    x = jnp.maximum((x_ref[0].reshape(G * N, D) + agg) @ wg[...] + bg[...],
                    0.0)
    out_ref[0] = _attn_ffn(x, wq[...], wk[...], wv[...], bv[...], wo[...],
                           w1[...], w2[...], l1s[...], l1b[...], l2s[...],
                           l2b[...]).reshape(G, N, D)


def _body_last(x_ref, a0_ref, a1_ref, wg, bg, out_ref):
    agg = jnp.concatenate([a0_ref[0], a1_ref[0]], axis=-1).reshape(G * N, D)
    x = jnp.maximum((x_ref[0].reshape(G * N, D) + agg) @ wg[...] + bg[...],
                    0.0)
    out_ref[0] = x.reshape(G, N, D)


_XSPEC = pl.BlockSpec((1, G, N, D), lambda b: (b, 0, 0, 0))
_WSPEC = pl.BlockSpec((D, D), lambda b: (0, 0))
_VSPEC = pl.BlockSpec((1, D), lambda b: (0, 0))
_HSPEC = pl.BlockSpec((1, G, N, HW), lambda b: (b, 0, 0, 0))
DV = NHEAD * (DH + 1)
_WVSPEC = pl.BlockSpec((D, DV), lambda b: (0, 0))
_BVSPEC = pl.BlockSpec((1, DV), lambda b: (0, 0))
_OUTSHAPE = jax.ShapeDtypeStruct((B // G, G, N, D), jnp.float32)
_LWSPECS = [_WSPEC, _WSPEC, _WVSPEC, _BVSPEC, _WSPEC, _WSPEC, _WSPEC,
            _VSPEC, _VSPEC, _VSPEC, _VSPEC]


def _tc_first(lab3, emb_p, *lw):
    return pl.pallas_call(
        _body_first,
        grid=(B // G,),
        in_specs=[pl.BlockSpec((1, G, N), lambda b: (b, 0, 0)),
                  pl.BlockSpec((VPAD, D), lambda b: (0, 0))] + _LWSPECS,
        out_specs=_XSPEC,
        out_shape=_OUTSHAPE,
    )(lab3, emb_p, *lw)


def _tc_mid(x, a0, a1, wg, bg, *lw):
    return pl.pallas_call(
        _body_mid,
        grid=(B // G,),
        in_specs=[_XSPEC, _HSPEC, _HSPEC, _WSPEC, _VSPEC] + _LWSPECS,
        out_specs=_XSPEC,
        out_shape=_OUTSHAPE,
    )(x, a0, a1, wg, bg, *lw)


def _tc_last(x, a0, a1, wg, bg):
    return pl.pallas_call(
        _body_last,
        grid=(B // G,),
        in_specs=[_XSPEC, _HSPEC, _HSPEC, _WSPEC, _VSPEC],
        out_specs=_XSPEC,
        out_shape=_OUTSHAPE,
    )(x, a0, a1, wg, bg)


def _sc_agg_kernel(x_hbm, eidx_hbm, zeros_hbm, outT_hbm,
                   x_v, agg_v, src_v, dst_v, sb_v, db_v, sem):
    cid = lax.axis_index("c")
    sid = lax.axis_index("s")
    wid = cid * 16 + sid
    lane = lax.iota(jnp.int32, 16)

    xcp = pltpu.async_copy(x_hbm.at[pl.ds(wid * N, N)], x_v, sem)
    pltpu.sync_copy(eidx_hbm.at[pl.ds(2 * wid * EPAD, EPAD)], src_v)
    pltpu.sync_copy(eidx_hbm.at[pl.ds((2 * wid + 1) * EPAD, EPAD)], dst_v)

    for g in range(EPAD // 16):
        sb_v[pl.ds(g * 16, 16)] = src_v[pl.ds(g * 16, 16)]
        db_v[pl.ds(g * 16, 16)] = dst_v[pl.ds(g * 16, 16)]

    tail_mask = lane < (E - (EPAD // 16 - 1) * 16)

    first = True
    for h in range(2):
        pltpu.sync_copy(zeros_hbm, agg_v)
        if first:
            xcp.wait()
            first = False

        for g in range(EPAD // 16):
            sb = sb_v[pl.ds(g * 16, 16)]
            db = db_v[pl.ds(g * 16, 16)]
            mask = None if g < EPAD // 16 - 1 else tail_mask

            def body(_, fvec, x_v=x_v, agg_v=agg_v, sb=sb, db=db,
                     mask=mask):
                vals = []
                cols = []
                for u in range(UNROLL):
                    m = (fvec + u) & (HW - 1)
                    vals.append(plsc.load_gather(x_v, [sb, m + h * HW],
                                                 mask=mask))
                    cols.append(m)
                for u in range(UNROLL):
                    plsc.addupdate_scatter(agg_v, [db, cols[u]], vals[u],
                                           mask=mask)
                return fvec + UNROLL

            lax.fori_loop(0, HW // UNROLL, body, lane)
        pltpu.sync_copy(agg_v,
                        outT_hbm.at[pl.ds((h * B + wid) * N, N)])


def _sc_agg(x2d, eidx1, zeros):
    mesh = plsc.VectorSubcoreMesh(core_axis_name="c", subcore_axis_name="s")
    run = pl.kernel(
        _sc_agg_kernel,
        out_type=jax.ShapeDtypeStruct((2 * B * N, HW), jnp.float32),
        mesh=mesh,
        scratch_types=[
            pltpu.VMEM((N, D), jnp.float32),
            pltpu.VMEM((N, HW), jnp.float32),
            pltpu.VMEM((EPAD,), jnp.int32),
            pltpu.VMEM((EPAD,), jnp.int32),
            pltpu.VMEM((EPAD,), jnp.int32),
            pltpu.VMEM((EPAD,), jnp.int32),
            pltpu.SemaphoreType.DMA,
        ],
        compiler_params=pltpu.CompilerParams(needs_layout_passes=False,
                                             use_tc_tiling_on_sc=True),
    )
    return run(x2d, eidx1, zeros)


def kernel(node_labels, edge_index, params):
    emb_p = jnp.zeros((VPAD, D), jnp.float32).at[:VOCAB].set(params["emb"])
    lab3 = node_labels.reshape(B // G, G, N).astype(jnp.int32)
    pad = jnp.broadcast_to(jnp.array([[0], [N]], jnp.int32), (B, 2, EPAD - E))
    eidx1 = jnp.concatenate([edge_index.astype(jnp.int32), pad],
                            axis=2).reshape(B * 2 * EPAD)
    zeros = jnp.zeros((N, HW), jnp.float32)

    layers = params["layers"]

    qscale = math.log2(math.e) / math.sqrt(DH)

    def lw(l):
        p = layers[l]
        wv_ext = jnp.pad(p["Wv"].reshape(D, NHEAD, DH),
                         ((0, 0), (0, 0), (0, 1))).reshape(D, DV)
        bv_ext = jnp.pad(jnp.zeros((NHEAD, DH), jnp.float32),
                         ((0, 0), (0, 1)),
                         constant_values=1.0).reshape(1, DV)
        return [p["Wq"] * qscale, p["Wk"], wv_ext, bv_ext, p["Wo"], p["W1"],
                p["W2"],
                p["ln1_s"].reshape(1, D), p["ln1_b"].reshape(1, D),
                p["ln2_s"].reshape(1, D), p["ln2_b"].reshape(1, D)]

    x = _tc_first(lab3, emb_p, *lw(0))
    for l in range(NLAYERS):
        agg5 = _sc_agg(x.reshape(B * N, D), eidx1, zeros).reshape(
            2, B // G, G, N, HW)
        wg = layers[l]["Wg"]
        bg = layers[l]["bg"].reshape(1, D)
        if l < NLAYERS - 1:
            x = _tc_mid(x, agg5[0], agg5[1], wg, bg, *lw(l + 1))
        else:
            x = _tc_last(x, agg5[0], agg5[1], wg, bg)
    return x.reshape(B, N, D)


# speedup vs baseline: 1.0073x; 1.0073x over previous
import functools
import math

import jax
import jax.numpy as jnp
from jax import lax
from jax.experimental import pallas as pl
from jax.experimental.pallas import tpu as pltpu
from jax.experimental.pallas import tpu_sc as plsc

D = 256
NHEAD = 32
DH = D // NHEAD
NLAYERS = 3
VOCAB = 55
VPAD = 64
B, N, E = 32, 256, 510
EPAD = 512
XW = N * D
HW = D // 2
AGW = N * HW
UNROLL = 8


def _ln(x, s, b, eps=1e-5):
    mu = jnp.mean(x, axis=-1, keepdims=True)
    var = jnp.mean((x - mu) ** 2, axis=-1, keepdims=True)
    return (x - mu) / jnp.sqrt(var + eps) * s + b


def _attn_ffn(x, wq, wk, wv, bv, wo, w1, w2, l1s, l1b, l2s, l2b):
    q = (x @ wq).astype(jnp.bfloat16)
    k = (x @ wk).astype(jnp.bfloat16)
    v = (x @ wv + bv).astype(jnp.bfloat16)
    pairs = [(g, h) for g in range(G) for h in range(NHEAD)]
    outs = {}

    def _score(g, h):
        rows = slice(g * N, (g + 1) * N)
        sl = slice(h * DH, (h + 1) * DH)
        return lax.dot_general(q[rows, sl], k[rows, sl],
                               (((1,), (1,)), ((), ())),
                               preferred_element_type=jnp.float32)

    def _finish(g, h, s):
        rows = slice(g * N, (g + 1) * N)
        sl = slice(h * (DH + 1), (h + 1) * (DH + 1))
        p = jnp.exp2(s.astype(jnp.bfloat16))
        outs[(g, h)] = lax.dot_general(p, v[rows, sl],
                                       (((1,), (0,)), ((), ())),
                                       preferred_element_type=jnp.float32)

    prev = None
    for g, h in pairs:
        s = _score(g, h)
        if prev is not None:
            _finish(*prev)
        prev = (g, h, s)
    _finish(*prev)

    expander = (lax.broadcasted_iota(jnp.int32, (NHEAD, D), 0)
                == (lax.broadcasted_iota(jnp.int32, (NHEAD, D), 1) >> 3)
                ).astype(jnp.float32)
    gouts = []
    for g in range(G):
        og = jnp.concatenate(
            [outs[(g, h)][:, :DH] for h in range(NHEAD)], axis=1)
        sg = jnp.concatenate(
            [outs[(g, h)][:, DH:DH + 1] for h in range(NHEAD)], axis=1)
        gouts.append(og * ((1.0 / sg) @ expander))
    o = jnp.concatenate(gouts, axis=0) @ wo
    x = _ln(x + o, l1s, l1b)
    ff = jnp.maximum(x @ w1, 0.0) @ w2
    x = _ln(x + ff, l2s, l2b)
    return x


G = 2


def _body_first(lab_ref, emb_ref, wq, wk, wv, bv, wo, w1, w2, l1s, l1b, l2s,
                l2b, out_ref):
    iota = lax.broadcasted_iota(jnp.int32, (N, VPAD), 1)
    onehot = jnp.concatenate(
        [(lab_ref[0, g, :][:, None] == iota).astype(jnp.float32)
         for g in range(G)], axis=0)
    x = onehot @ emb_ref[...]
    out_ref[0] = _attn_ffn(x, wq[...], wk[...], wv[...], bv[...], wo[...],
                           w1[...], w2[...], l1s[...], l1b[...], l2s[...],
                           l2b[...]).reshape(G, N, D)


def _body_mid(x_ref, a0_ref, a1_ref, wg, bg, wq, wk, wv, bv, wo, w1, w2, l1s,
              l1b, l2s, l2b, out_ref):
    agg = jnp.concatenate([a0_ref[0], a1_ref[0]], axis=-1).reshape(G * N, D)
    x = jnp.maximum((x_ref[0].reshape(G * N, D) + agg) @ wg[...] + bg[...],
                    0.0)
    out_ref[0] = _attn_ffn(x, wq[...], wk[...], wv[...], bv[...], wo[...],
                           w1[...], w2[...], l1s[...], l1b[...], l2s[...],
                           l2b[...]).reshape(G, N, D)


def _body_last(x_ref, a0_ref, a1_ref, wg, bg, out_ref):
    agg = jnp.concatenate([a0_ref[0], a1_ref[0]], axis=-1).reshape(G * N, D)
    x = jnp.maximum((x_ref[0].reshape(G * N, D) + agg) @ wg[...] + bg[...],
                    0.0)
    out_ref[0] = x.reshape(G, N, D)


_XSPEC = pl.BlockSpec((1, G, N, D), lambda b: (b, 0, 0, 0))
_WSPEC = pl.BlockSpec((D, D), lambda b: (0, 0))
_VSPEC = pl.BlockSpec((1, D), lambda b: (0, 0))
_HSPEC = pl.BlockSpec((1, G, N, HW), lambda b: (b, 0, 0, 0))
DV = NHEAD * (DH + 1)
_WVSPEC = pl.BlockSpec((D, DV), lambda b: (0, 0))
_BVSPEC = pl.BlockSpec((1, DV), lambda b: (0, 0))
_OUTSHAPE = jax.ShapeDtypeStruct((B // G, G, N, D), jnp.float32)
_LWSPECS = [_WSPEC, _WSPEC, _WVSPEC, _BVSPEC, _WSPEC, _WSPEC, _WSPEC,
            _VSPEC, _VSPEC, _VSPEC, _VSPEC]


def _tc_first(lab3, emb_p, *lw):
    return pl.pallas_call(
        _body_first,
        grid=(B // G,),
        in_specs=[pl.BlockSpec((1, G, N), lambda b: (b, 0, 0)),
                  pl.BlockSpec((VPAD, D), lambda b: (0, 0))] + _LWSPECS,
        out_specs=_XSPEC,
        out_shape=_OUTSHAPE,
    )(lab3, emb_p, *lw)


def _tc_mid(x, a0, a1, wg, bg, *lw):
    return pl.pallas_call(
        _body_mid,
        grid=(B // G,),
        in_specs=[_XSPEC, _HSPEC, _HSPEC, _WSPEC, _VSPEC] + _LWSPECS,
        out_specs=_XSPEC,
        out_shape=_OUTSHAPE,
    )(x, a0, a1, wg, bg, *lw)


def _tc_last(x, a0, a1, wg, bg):
    return pl.pallas_call(
        _body_last,
        grid=(B // G,),
        in_specs=[_XSPEC, _HSPEC, _HSPEC, _WSPEC, _VSPEC],
        out_specs=_XSPEC,
        out_shape=_OUTSHAPE,
    )(x, a0, a1, wg, bg)


def _sc_agg_kernel(x_hbm, eidx_hbm, zeros_hbm, outT_hbm,
                   x_v, agg_v, src_v, dst_v, sb_v, db_v, sem):
    cid = lax.axis_index("c")
    sid = lax.axis_index("s")
    wid = cid * 16 + sid
    lane = lax.iota(jnp.int32, 16)

    xcp = pltpu.async_copy(x_hbm.at[pl.ds(wid * N, N)], x_v, sem)
    pltpu.sync_copy(eidx_hbm.at[pl.ds(2 * wid * EPAD, EPAD)], src_v)
    pltpu.sync_copy(eidx_hbm.at[pl.ds((2 * wid + 1) * EPAD, EPAD)], dst_v)

    for g in range(EPAD // 16):
        sb_v[pl.ds(g * 16, 16)] = src_v[pl.ds(g * 16, 16)]
        db_v[pl.ds(g * 16, 16)] = dst_v[pl.ds(g * 16, 16)]

    tail_mask = lane < (E - (EPAD // 16 - 1) * 16)

    first = True
    for h in range(2):
        pltpu.sync_copy(zeros_hbm, agg_v)
        if first:
            xcp.wait()
            first = False

        for g in range(EPAD // 16):
            sb = sb_v[pl.ds(g * 16, 16)]
            db = db_v[pl.ds(g * 16, 16)]
            mask = None if g < EPAD // 16 - 1 else tail_mask

            def body(_, fvec, x_v=x_v, agg_v=agg_v, sb=sb, db=db,
                     mask=mask):
                vals = []
                cols = []
                for u in range(UNROLL):
                    m = (fvec + u) & (HW - 1)
                    vals.append(plsc.load_gather(x_v, [sb, m + h * HW],
                                                 mask=mask))
                    cols.append(m)
                for u in range(UNROLL):
                    plsc.addupdate_scatter(agg_v, [db, cols[u]], vals[u],
                                           mask=mask)
                return fvec + UNROLL

            lax.fori_loop(0, HW // UNROLL, body, lane)
        pltpu.sync_copy(agg_v,
                        outT_hbm.at[pl.ds((h * B + wid) * N, N)])


def _sc_agg(x2d, eidx1, zeros):
    mesh = plsc.VectorSubcoreMesh(core_axis_name="c", subcore_axis_name="s")
    run = pl.kernel(
        _sc_agg_kernel,
        out_type=jax.ShapeDtypeStruct((2 * B * N, HW), jnp.float32),
        mesh=mesh,
        scratch_types=[
            pltpu.VMEM((N, D), jnp.float32),
            pltpu.VMEM((N, HW), jnp.float32),
            pltpu.VMEM((EPAD,), jnp.int32),
            pltpu.VMEM((EPAD,), jnp.int32),
            pltpu.VMEM((EPAD,), jnp.int32),
            pltpu.VMEM((EPAD,), jnp.int32),
            pltpu.SemaphoreType.DMA,
        ],
        compiler_params=pltpu.CompilerParams(needs_layout_passes=False,
                                             use_tc_tiling_on_sc=True),
    )
    return run(x2d, eidx1, zeros)


def kernel(node_labels, edge_index, params):
    emb_p = jnp.zeros((VPAD, D), jnp.float32).at[:VOCAB].set(params["emb"])
    lab3 = node_labels.reshape(B // G, G, N).astype(jnp.int32)
    pad = jnp.broadcast_to(jnp.array([[0], [N]], jnp.int32), (B, 2, EPAD - E))
    eidx1 = jnp.concatenate([edge_index.astype(jnp.int32), pad],
                            axis=2).reshape(B * 2 * EPAD)
    zeros = jnp.zeros((N, HW), jnp.float32)

    layers = params["layers"]

    qscale = math.log2(math.e) / math.sqrt(DH)

    def lw(l):
        p = layers[l]
        wv_ext = jnp.pad(p["Wv"].reshape(D, NHEAD, DH),
                         ((0, 0), (0, 0), (0, 1))).reshape(D, DV)
        bv_ext = jnp.pad(jnp.zeros((NHEAD, DH), jnp.float32),
                         ((0, 0), (0, 1)),
                         constant_values=1.0).reshape(1, DV)
        return [p["Wq"] * qscale, p["Wk"], wv_ext, bv_ext, p["Wo"], p["W1"],
                p["W2"],
                p["ln1_s"].reshape(1, D), p["ln1_b"].reshape(1, D),
                p["ln2_s"].reshape(1, D), p["ln2_b"].reshape(1, D)]

    x = _tc_first(lab3, emb_p, *lw(0))
    for l in range(NLAYERS):
        agg5 = _sc_agg(x.reshape(B * N, D), eidx1, zeros).reshape(
            2, B // G, G, N, HW)
        wg = layers[l]["Wg"]
        bg = layers[l]["bg"].reshape(1, D)
        if l < NLAYERS - 1:
            x = _tc_mid(x, agg5[0], agg5[1], wg, bg, *lw(l + 1))
        else:
            x = _tc_last(x, agg5[0], agg5[1], wg, bg)
    return x.reshape(B, N, D)
